# baseline (speedup 1.0000x reference)
import functools

import jax
import jax.numpy as jnp
from jax import lax
from jax.experimental import pallas as pl
from jax.experimental.pallas import tpu as pltpu
from jax.experimental.pallas import tpu_sc as plsc

B = 16384
D = 16
NC = 2
NS = 16
NW = NC * NS
BPW = B // NW
CH = 128
NCH = BPW // CH


def _sc_gather(user_indices, item_indices, mf_emb_user, mf_emb_item,
               mlp_emb_user, mlp_emb_item):
    mesh = plsc.VectorSubcoreMesh(core_axis_name="c", subcore_axis_name="s")

    @functools.partial(
        pl.kernel,
        out_type=[jax.ShapeDtypeStruct((B, D), jnp.float32) for _ in range(4)],
        mesh=mesh,
        scratch_types=[
            pltpu.VMEM((BPW,), jnp.int32),
            pltpu.VMEM((BPW,), jnp.int32),
            pltpu.VMEM((BPW, D), jnp.float32),
            pltpu.VMEM((BPW, D), jnp.float32),
            pltpu.VMEM((BPW, D), jnp.float32),
            pltpu.VMEM((BPW, D), jnp.float32),
            pltpu.SemaphoreType.DMA,
        ],
        compiler_params=pltpu.CompilerParams(use_tc_tiling_on_sc=False),
    )
    def gather_k(uidx, iidx, t_mfu, t_mfi, t_mlu, t_mli,
                 o_mfu, o_mfi, o_mlu, o_mli,
                 uix, iix, r_mfu, r_mfi, r_mlu, r_mli, sem):
        wid = lax.axis_index("s") * NC + lax.axis_index("c")
        base = wid * BPW
        pltpu.sync_copy(uidx.at[pl.ds(base, BPW)], uix)
        pltpu.sync_copy(iidx.at[pl.ds(base, BPW)], iix)
        handles = []
        for tbl, ix, rbuf in ((t_mfu, uix, r_mfu), (t_mfi, iix, r_mfi),
                              (t_mlu, uix, r_mlu), (t_mli, iix, r_mli)):
            for c in range(NCH):
                sl = pl.ds(c * CH, CH)
                handles.append(
                    pltpu.async_copy(tbl.at[ix.at[sl]], rbuf.at[sl], sem))
        for h in handles:
            h.wait()
        out_sl = pl.ds(base, BPW)
        pltpu.sync_copy(r_mfu, o_mfu.at[out_sl])
        pltpu.sync_copy(r_mfi, o_mfi.at[out_sl])
        pltpu.sync_copy(r_mlu, o_mlu.at[out_sl])
        pltpu.sync_copy(r_mli, o_mli.at[out_sl])

    return gather_k(user_indices, item_indices, mf_emb_user, mf_emb_item,
                    mlp_emb_user, mlp_emb_item)


BB = 2048


def _dense_body(mfu_ref, mfi_ref, mlu_ref, mli_ref, w0t_ref, b0_ref, wp_ref,
                bp_ref, out_ref):
    mf = mfu_ref[...] * mfi_ref[...]
    mlp_vec = jnp.concatenate([mlu_ref[...], mli_ref[...]], axis=1)
    h = jnp.dot(mlp_vec, w0t_ref[...], preferred_element_type=jnp.float32)
    h = jnp.maximum(h + b0_ref[...], 0.0)
    wp = wp_ref[...]
    logit = (jnp.sum(mf * wp[:, :D], axis=1)
             + jnp.sum(h * wp[:, D:], axis=1)
             + bp_ref[0, 0])
    out_ref[...] = jax.nn.sigmoid(logit).reshape(1, 1, BB)


def _tc_dense(mfu, mfi, mlu, mli, W0, b0, Wp, bp):
    nblk = B // BB
    row_spec = pl.BlockSpec((BB, D), lambda i: (i, 0))
    full = lambda shape: pl.BlockSpec(shape, lambda i: (0,) * len(shape))
    out2d = pl.pallas_call(
        _dense_body,
        grid=(nblk,),
        in_specs=[row_spec, row_spec, row_spec, row_spec,
                  full((2 * D, D)), full((1, D)), full((1, 2 * D)),
                  full((1, 1))],
        out_specs=pl.BlockSpec((1, 1, BB), lambda i: (i, 0, 0)),
        out_shape=jax.ShapeDtypeStruct((nblk, 1, BB), jnp.float32),
    )(mfu, mfi, mlu, mli, W0.T, b0.reshape(1, D), Wp, bp.reshape(1, 1))
    return out2d.reshape(B)


def kernel(user_indices, item_indices, mf_emb_user, mf_emb_item,
           mlp_emb_user, mlp_emb_item, W0, b0, Wp, bp):
    mfu, mfi, mlu, mli = _sc_gather(
        user_indices.astype(jnp.int32), item_indices.astype(jnp.int32),
        mf_emb_user, mf_emb_item, mlp_emb_user, mlp_emb_item)
    return _tc_dense(mfu, mfi, mlu, mli, W0, b0, Wp, bp)
